# baseline (speedup 1.0000x reference)
import functools

import jax
import jax.numpy as jnp
from jax import lax
from jax.experimental import pallas as pl
from jax.experimental.pallas import tpu as pltpu
from jax.experimental.pallas import tpu_sc as plsc

NUM_USERS = 1000000
NUM_ITEMS = 1000000
LATENT_DIM = 32
BATCH = 16384

_INFO = plsc.get_sparse_core_info()
NC, NS, L = _INFO.num_cores, _INFO.num_subcores, _INFO.num_lanes
NW = NC * NS
BPW = BATCH // NW
GROUPS = BPW // L

_MESH = plsc.VectorSubcoreMesh(core_axis_name="c", subcore_axis_name="s")


@functools.partial(
    pl.kernel,
    mesh=_MESH,
    compiler_params=pltpu.CompilerParams(needs_layout_passes=False,
                                         use_tc_tiling_on_sc=False),
    out_type=jax.ShapeDtypeStruct((BATCH,), jnp.float32),
    scratch_types=[
        pltpu.VMEM((BPW,), jnp.int32),
        pltpu.VMEM((BPW,), jnp.int32),
        pltpu.VMEM((BPW, LATENT_DIM), jnp.float32),
        pltpu.VMEM((BPW, LATENT_DIM), jnp.float32),
        pltpu.VMEM((BPW,), jnp.float32),
        pltpu.VMEM((L * L,), jnp.float32),
        pltpu.VMEM((L * L,), jnp.float32),
        pltpu.SemaphoreType.DMA,
        pltpu.SemaphoreType.DMA,
    ],
)
def _var_mf_sc(users_h, items_h, ut_h, it_h, out_h,
               uidx_v, iidx_v, urows_v, irows_v, gout_v, nbuf_v, dbuf_v,
               sem_u, sem_i):
    wid = lax.axis_index("s") * NC + lax.axis_index("c")
    base = wid * BPW

    pltpu.sync_copy(users_h.at[pl.ds(base, BPW)], uidx_v)
    pltpu.sync_copy(items_h.at[pl.ds(base, BPW)], iidx_v)
    cp_u = pltpu.async_copy(ut_h.at[uidx_v], urows_v, sem_u)
    cp_i = pltpu.async_copy(it_h.at[iidx_v], irows_v, sem_i)
    cp_u.wait()
    cp_i.wait()

    lane = lax.iota(jnp.int32, L)

    def group(g, _):
        for r in range(L):
            row = g * L + r
            u0 = urows_v[row, pl.ds(0, L)]
            u1 = urows_v[row, pl.ds(L, L)]
            i0 = irows_v[row, pl.ds(0, L)]
            i1 = irows_v[row, pl.ds(L, L)]
            e0 = jnp.exp(i0)
            e1 = jnp.exp(i1)
            s0 = 1.0 / (1.0 + jnp.exp(-u0))
            s1 = 1.0 / (1.0 + jnp.exp(-u1))
            nbuf_v[pl.ds(r * L, L)] = e0 * s0 + e1 * s1
            dbuf_v[pl.ds(r * L, L)] = e0 + e1
        num = jnp.zeros((L,), jnp.float32)
        den = jnp.zeros((L,), jnp.float32)
        rowbase = lane * L
        for k in range(L):
            num = num + plsc.load_gather(nbuf_v, [rowbase + k])
            den = den + plsc.load_gather(dbuf_v, [rowbase + k])
        gout_v[pl.ds(g * L, L)] = num / den
        return 0

    lax.fori_loop(0, GROUPS, group, 0)

    pltpu.sync_copy(gout_v, out_h.at[pl.ds(base, BPW)])


def kernel(users, items, user_table, item_table):
    users = users.astype(jnp.int32)
    items = items.astype(jnp.int32)
    return _var_mf_sc(users, items, user_table, item_table)
